# baseline (speedup 1.0000x reference)
import jax
import jax.numpy as jnp
from jax.experimental import pallas as pl
from jax.experimental.pallas import tpu as pltpu

_B = 8
_P = 256
_D = 384
_M = 20000
_G = 128
_NB = 10
_MBLK = _M // _NB
_NCH = 4
_CHUNK = _MBLK // _NCH
_ALPHA = 0.7


def _body(q_ref, g_ref, mbg_ref, mb_hbm, out_ref, qn_s, a2_s, m_s):
    q = q_ref[...]
    nrm = jnp.sqrt(jnp.sum(q * q, axis=1, keepdims=True))
    qn = q / (nrm + 1e-12)
    qn_s[...] = qn.astype(jnp.bfloat16)
    a2_s[...] = jnp.sum(qn * qn, axis=1, keepdims=True)
    m_s[...] = jnp.full((_B * _P, 1), jnp.inf, jnp.float32)

    def _step(mb_ref):
        mb = mb_ref[...]
        b2c = jnp.sum(mb * mb, axis=1, keepdims=True) * 0.5
        mbb = mb.astype(jnp.bfloat16)
        qn = qn_s[...]
        m = m_s[...]
        for c in range(_NCH):
            sl = slice(c * _CHUNK, (c + 1) * _CHUNK)
            b2h = jax.lax.transpose(b2c[sl, :], (1, 0))
            t = jax.lax.dot_general(qn, mbb[sl, :],
                                    (((1,), (1,)), ((), ())),
                                    preferred_element_type=jnp.float32)
            m = jnp.minimum(m, jnp.min(b2h - t, axis=1, keepdims=True))
        m_s[...] = m

    pltpu.emit_pipeline(
        _step,
        grid=(_NB,),
        in_specs=[pl.BlockSpec((_MBLK, _D), lambda nb: (nb, 0))],
    )(mb_hbm)

    d2 = a2_s[...] + 2.0 * m_s[...]
    g = g_ref[...]
    gn = g / (jnp.sqrt(jnp.sum(g * g, axis=1, keepdims=True)) + 1e-12)
    gsq = jnp.sum(gn * gn, axis=1, keepdims=True)
    mbg = mbg_ref[...]
    bg2 = jax.lax.transpose(
        jnp.sum(mbg * mbg, axis=1, keepdims=True), (1, 0))
    tg = jax.lax.dot_general(gn, mbg, (((1,), (1,)), ((), ())),
                             preferred_element_type=jnp.float32)
    gmin = jnp.min(bg2 - 2.0 * tg, axis=1, keepdims=True) + gsq
    gd = jnp.sqrt(jnp.maximum(gmin, 0.0))
    for b in range(_B):
        d2max = jnp.max(d2[b * _P:(b + 1) * _P, :])
        local = jnp.sqrt(jnp.maximum(d2max, 0.0))
        out_ref[b] = _ALPHA * local + (1.0 - _ALPHA) * gd[b, 0]


def kernel(patches, global_feat, mb_local, mb_global):
    q = patches.reshape(_B * _P, _D)
    return pl.pallas_call(
        _body,
        in_specs=[
            pl.BlockSpec((_B * _P, _D), lambda: (0, 0)),
            pl.BlockSpec((_B, _D), lambda: (0, 0)),
            pl.BlockSpec((_G, _D), lambda: (0, 0)),
            pl.BlockSpec(memory_space=pl.ANY),
        ],
        out_specs=pl.BlockSpec(memory_space=pltpu.SMEM),
        out_shape=jax.ShapeDtypeStruct((_B,), jnp.float32),
        scratch_shapes=[
            pltpu.VMEM((_B * _P, _D), jnp.bfloat16),
            pltpu.VMEM((_B * _P, 1), jnp.float32),
            pltpu.VMEM((_B * _P, 1), jnp.float32),
        ],
    )(q, global_feat, mb_global, mb_local)


# speedup vs baseline: 1.0091x; 1.0091x over previous
import jax
import jax.numpy as jnp
from jax.experimental import pallas as pl
from jax.experimental.pallas import tpu as pltpu

_B = 8
_P = 256
_D = 384
_M = 20000
_G = 128
_NB = 5
_MBLK = _M // _NB
_NCH = 8
_CHUNK = _MBLK // _NCH
_ALPHA = 0.7


def _body(q_ref, g_ref, mbg_ref, mb_hbm, out_ref, qn_s, a2_s, m_s):
    q = q_ref[...]
    nrm = jnp.sqrt(jnp.sum(q * q, axis=1, keepdims=True))
    qn = q / (nrm + 1e-12)
    qn_s[...] = qn.astype(jnp.bfloat16)
    a2_s[...] = jnp.sum(qn * qn, axis=1, keepdims=True)
    m_s[...] = jnp.full((_B * _P, 1), jnp.inf, jnp.float32)

    def _step(mb_ref):
        mb = mb_ref[...]
        b2c = jnp.sum(mb * mb, axis=1, keepdims=True) * 0.5
        mbb = mb.astype(jnp.bfloat16)
        qn = qn_s[...]
        m = m_s[...]
        for c in range(_NCH):
            sl = slice(c * _CHUNK, (c + 1) * _CHUNK)
            b2h = jax.lax.transpose(b2c[sl, :], (1, 0))
            t = jax.lax.dot_general(qn, mbb[sl, :],
                                    (((1,), (1,)), ((), ())),
                                    preferred_element_type=jnp.float32)
            m = jnp.minimum(m, jnp.min(b2h - t, axis=1, keepdims=True))
        m_s[...] = m

    pltpu.emit_pipeline(
        _step,
        grid=(_NB,),
        in_specs=[pl.BlockSpec((_MBLK, _D), lambda nb: (nb, 0))],
    )(mb_hbm)

    d2 = a2_s[...] + 2.0 * m_s[...]
    g = g_ref[...]
    gn = g / (jnp.sqrt(jnp.sum(g * g, axis=1, keepdims=True)) + 1e-12)
    gsq = jnp.sum(gn * gn, axis=1, keepdims=True)
    mbg = mbg_ref[...]
    bg2 = jax.lax.transpose(
        jnp.sum(mbg * mbg, axis=1, keepdims=True), (1, 0))
    tg = jax.lax.dot_general(gn, mbg, (((1,), (1,)), ((), ())),
                             preferred_element_type=jnp.float32)
    gmin = jnp.min(bg2 - 2.0 * tg, axis=1, keepdims=True) + gsq
    gd = jnp.sqrt(jnp.maximum(gmin, 0.0))
    for b in range(_B):
        d2max = jnp.max(d2[b * _P:(b + 1) * _P, :])
        local = jnp.sqrt(jnp.maximum(d2max, 0.0))
        out_ref[b] = _ALPHA * local + (1.0 - _ALPHA) * gd[b, 0]


def kernel(patches, global_feat, mb_local, mb_global):
    q = patches.reshape(_B * _P, _D)
    return pl.pallas_call(
        _body,
        in_specs=[
            pl.BlockSpec((_B * _P, _D), lambda: (0, 0)),
            pl.BlockSpec((_B, _D), lambda: (0, 0)),
            pl.BlockSpec((_G, _D), lambda: (0, 0)),
            pl.BlockSpec(memory_space=pl.ANY),
        ],
        out_specs=pl.BlockSpec(memory_space=pltpu.SMEM),
        out_shape=jax.ShapeDtypeStruct((_B,), jnp.float32),
        scratch_shapes=[
            pltpu.VMEM((_B * _P, _D), jnp.bfloat16),
            pltpu.VMEM((_B * _P, 1), jnp.float32),
            pltpu.VMEM((_B * _P, 1), jnp.float32),
        ],
    )(q, global_feat, mb_global, mb_local)
